# Initial kernel scaffold; baseline (speedup 1.0000x reference)
import functools

import jax
import jax.numpy as jnp
from jax import lax
from jax.experimental import pallas as pl
from jax.experimental.pallas import tpu as pltpu
from jax.experimental.pallas import tpu_sc as plsc

NUM_CORES = 2
NUM_SUBCORES = 16
NUM_WORKERS = NUM_CORES * NUM_SUBCORES
CHUNK = 3200


@functools.partial(jax.jit, static_argnames=("total", "dim"))
def _gather_flat(x_flat, embedding, *, total, dim):
    per_worker = total // NUM_WORKERS
    n_chunks = per_worker // CHUNK

    mesh = plsc.VectorSubcoreMesh(
        core_axis_name="c", subcore_axis_name="s",
        num_cores=NUM_CORES, num_subcores=NUM_SUBCORES,
    )

    @functools.partial(
        pl.kernel,
        out_type=jax.ShapeDtypeStruct((total, dim), jnp.float32),
        mesh=mesh,
        scratch_types=[
            pltpu.VMEM((CHUNK,), jnp.int32),
            pltpu.VMEM((CHUNK, dim), jnp.float32),
            pltpu.SemaphoreType.DMA,
        ],
    )
    def k(x_hbm, table_hbm, out_hbm, idx_v, rows_v, sem):
        wid = lax.axis_index("s") * NUM_CORES + lax.axis_index("c")
        base = wid * per_worker

        def chunk_body(i, carry):
            off = base + i * CHUNK
            pltpu.sync_copy(x_hbm.at[pl.ds(off, CHUNK)], idx_v)
            pltpu.async_copy(table_hbm.at[idx_v], rows_v, sem).wait()
            pltpu.sync_copy(rows_v, out_hbm.at[pl.ds(off, CHUNK)])
            return carry

        lax.fori_loop(0, n_chunks, chunk_body, 0)

    return k(x_flat, embedding)


def kernel(x, embedding):
    total = x.shape[0] * x.shape[1]
    dim = embedding.shape[1]
    x_flat = x.reshape(total).astype(jnp.int32)
    out = _gather_flat(x_flat, embedding, total=total, dim=dim)
    return out.reshape(x.shape[0], x.shape[1], dim)


# speedup vs baseline: 1.1098x; 1.1098x over previous
import functools

import jax
import jax.numpy as jnp
from jax import lax
from jax.experimental import pallas as pl
from jax.experimental.pallas import tpu as pltpu
from jax.experimental.pallas import tpu_sc as plsc

NUM_CORES = 2
NUM_SUBCORES = 16
NUM_WORKERS = NUM_CORES * NUM_SUBCORES
CHUNK = 3200


@functools.partial(jax.jit, static_argnames=("total", "dim"))
def _gather_flat(x_flat, embedding, *, total, dim):
    per_worker = total // NUM_WORKERS
    n_chunks = per_worker // CHUNK

    mesh = plsc.VectorSubcoreMesh(
        core_axis_name="c", subcore_axis_name="s",
        num_cores=NUM_CORES, num_subcores=NUM_SUBCORES,
    )

    @functools.partial(
        pl.kernel,
        out_type=jax.ShapeDtypeStruct((total, dim), jnp.float32),
        mesh=mesh,
        scratch_types=[
            pltpu.VMEM((CHUNK,), jnp.int32),
            pltpu.VMEM((CHUNK, dim), jnp.float32),
            pltpu.SemaphoreType.DMA,
        ],
        compiler_params=pltpu.CompilerParams(use_tc_tiling_on_sc=False),
    )
    def k(x_hbm, table_hbm, out_hbm, idx_v, rows_v, sem):
        wid = lax.axis_index("s") * NUM_CORES + lax.axis_index("c")
        base = wid * per_worker

        def chunk_body(i, carry):
            off = base + i * CHUNK
            pltpu.sync_copy(x_hbm.at[pl.ds(off, CHUNK)], idx_v)
            pltpu.async_copy(table_hbm.at[idx_v], rows_v, sem).wait()
            pltpu.sync_copy(rows_v, out_hbm.at[pl.ds(off, CHUNK)])
            return carry

        lax.fori_loop(0, n_chunks, chunk_body, 0)

    return k(x_flat, embedding)


def kernel(x, embedding):
    total = x.shape[0] * x.shape[1]
    dim = embedding.shape[1]
    x_flat = x.reshape(total).astype(jnp.int32)
    out = _gather_flat(x_flat, embedding, total=total, dim=dim)
    return out.reshape(x.shape[0], x.shape[1], dim)


# speedup vs baseline: 1.1121x; 1.0021x over previous
import functools

import jax
import jax.numpy as jnp
from jax import lax
from jax.experimental import pallas as pl
from jax.experimental.pallas import tpu as pltpu
from jax.experimental.pallas import tpu_sc as plsc

NUM_CORES = 2
NUM_SUBCORES = 16
NUM_WORKERS = NUM_CORES * NUM_SUBCORES
CHUNK = 1600


@functools.partial(jax.jit, static_argnames=("total", "dim"))
def _gather_flat(x_flat, embedding, *, total, dim):
    per_worker = total // NUM_WORKERS
    n_chunks = per_worker // CHUNK

    mesh = plsc.VectorSubcoreMesh(
        core_axis_name="c", subcore_axis_name="s",
        num_cores=NUM_CORES, num_subcores=NUM_SUBCORES,
    )

    @functools.partial(
        pl.kernel,
        out_type=jax.ShapeDtypeStruct((total, dim), jnp.float32),
        mesh=mesh,
        scratch_types=[
            pltpu.VMEM((per_worker,), jnp.int32),
            pltpu.VMEM((CHUNK, dim), jnp.float32),
            pltpu.VMEM((CHUNK, dim), jnp.float32),
            pltpu.SemaphoreType.DMA,
            pltpu.SemaphoreType.DMA,
            pltpu.SemaphoreType.DMA,
            pltpu.SemaphoreType.DMA,
        ],
        compiler_params=pltpu.CompilerParams(use_tc_tiling_on_sc=False),
    )
    def k(x_hbm, table_hbm, out_hbm, idx_v, rows0, rows1, g0, g1, o0, o1):
        wid = lax.axis_index("s") * NUM_CORES + lax.axis_index("c")
        base = wid * per_worker
        pltpu.sync_copy(x_hbm.at[pl.ds(base, per_worker)], idx_v)

        rows = (rows0, rows1)
        gsem = (g0, g1)
        osem = (o0, o1)
        gwait = [None] * n_chunks
        owait = [None] * n_chunks

        def start_gather(i):
            b = i % 2
            gwait[i] = pltpu.async_copy(
                table_hbm.at[idx_v.at[pl.ds(i * CHUNK, CHUNK)]], rows[b], gsem[b]
            )

        start_gather(0)
        for i in range(n_chunks):
            b = i % 2
            if i + 1 < n_chunks:
                if i >= 1:
                    owait[i - 1].wait()
                start_gather(i + 1)
            gwait[i].wait()
            owait[i] = pltpu.async_copy(
                rows[b], out_hbm.at[pl.ds(base + i * CHUNK, CHUNK)], osem[b]
            )
        owait[n_chunks - 2].wait()
        owait[n_chunks - 1].wait()

    return k(x_flat, embedding)


def kernel(x, embedding):
    total = x.shape[0] * x.shape[1]
    dim = embedding.shape[1]
    x_flat = x.reshape(total).astype(jnp.int32)
    out = _gather_flat(x_flat, embedding, total=total, dim=dim)
    return out.reshape(x.shape[0], x.shape[1], dim)


# speedup vs baseline: 1.5775x; 1.4185x over previous
import functools

import jax
import jax.numpy as jnp
from jax import lax
from jax.experimental import pallas as pl
from jax.experimental.pallas import tpu as pltpu
from jax.experimental.pallas import tpu_sc as plsc

NUM_CORES = 2
NUM_SUBCORES = 16
NUM_WORKERS = NUM_CORES * NUM_SUBCORES
CHUNK = 256
LANES = 16


@functools.partial(jax.jit, static_argnames=("total",))
def _gather_flat(x_flat, table4, *, total):
    per_worker = total // NUM_WORKERS
    n_chunks = per_worker // CHUNK
    out_rows = total // 4
    rows_per_chunk = CHUNK // 4

    mesh = plsc.VectorSubcoreMesh(
        core_axis_name="c", subcore_axis_name="s",
        num_cores=NUM_CORES, num_subcores=NUM_SUBCORES,
    )

    @functools.partial(
        pl.kernel,
        out_type=jax.ShapeDtypeStruct((out_rows, 128), jnp.float32),
        mesh=mesh,
        scratch_types=[
            pltpu.VMEM((per_worker,), jnp.int32),
            pltpu.VMEM((CHUNK,), jnp.int32),
            pltpu.VMEM((CHUNK,), jnp.int32),
            pltpu.VMEM((CHUNK, 128), jnp.float32),
            pltpu.VMEM((CHUNK, 128), jnp.float32),
            pltpu.VMEM((CHUNK // 4, 128), jnp.float32),
            pltpu.VMEM((CHUNK // 4, 128), jnp.float32),
            pltpu.SemaphoreType.DMA,
            pltpu.SemaphoreType.DMA,
            pltpu.SemaphoreType.DMA,
            pltpu.SemaphoreType.DMA,
        ],
    )
    def k(x_hbm, table_hbm, out_hbm, idx_v, q0, q1, r0, r1, oc0, oc1,
          g0, g1, o0, o1):
        wid = lax.axis_index("s") * NUM_CORES + lax.axis_index("c")
        base = wid * per_worker
        out_base = wid * (per_worker // 4)
        pltpu.sync_copy(x_hbm.at[pl.ds(base, per_worker)], idx_v)

        qs = (q0, q1)
        rows = (r0, r1)
        outc = (oc0, oc1)
        gsem = (g0, g1)
        osem = (o0, o1)

        def fill_q(i, b):
            def body(j, carry):
                v = idx_v[pl.ds(i * CHUNK + j * LANES, LANES)]
                qs[b][pl.ds(j * LANES, LANES)] = lax.shift_right_logical(v, 2)
                return carry
            lax.fori_loop(0, CHUNK // LANES, body, 0, unroll=4)

        def start_gather(i, b):
            return pltpu.async_copy(table_hbm.at[qs[b]], rows[b], gsem[b])

        def wait_gather(b):
            pltpu.make_async_copy(table_hbm.at[qs[b]], rows[b], gsem[b]).wait()

        def wait_store(b):
            pltpu.make_async_copy(
                outc[b], out_hbm.at[pl.ds(out_base, rows_per_chunk)], osem[b]
            ).wait()

        def extract(i, b):
            def body(j, carry):
                vec = idx_v[pl.ds(i * CHUNK + j * LANES, LANES)]
                ovec = (vec & 3) * 32
                for rr in range(LANES):
                    r = j * LANES + rr
                    o = ovec[rr]
                    r4 = j * 4 + rr // 4
                    c0 = (rr % 4) * 32
                    outc[b][r4, pl.ds(c0, LANES)] = rows[b][r, pl.ds(o, LANES)]
                    outc[b][r4, pl.ds(c0 + LANES, LANES)] = (
                        rows[b][r, pl.ds(o + LANES, LANES)])
                return carry
            lax.fori_loop(0, CHUNK // LANES, body, 0)

        def start_store(i, b):
            return pltpu.async_copy(
                outc[b], out_hbm.at[pl.ds(out_base + i * rows_per_chunk,
                                          rows_per_chunk)], osem[b])

        fill_q(0, 0)
        start_gather(0, 0)
        fill_q(1, 1)
        start_gather(1, 1)

        for b in range(2):
            i = b
            wait_gather(b)
            extract(i, b)
            start_store(i, b)
            fill_q(i + 2, b)
            start_gather(i + 2, b)

        n_pairs = n_chunks // 2
        def pair_body(p, carry):
            for b in range(2):
                i = p * 2 + b
                wait_gather(b)
                wait_store(b)
                extract(i, b)
                start_store(i, b)
                fill_q(i + 2, b)
                start_gather(i + 2, b)
            return carry
        lax.fori_loop(1, n_pairs - 1, pair_body, 0)

        for b in range(2):
            i = n_chunks - 2 + b
            wait_gather(b)
            wait_store(b)
            extract(i, b)
            start_store(i, b)
        wait_store(0)
        wait_store(1)

    return k(x_flat, table4)


def kernel(x, embedding):
    total = x.shape[0] * x.shape[1]
    dim = embedding.shape[1]
    x_flat = x.reshape(total).astype(jnp.int32)
    table4 = embedding.reshape(-1, 128)
    out4 = _gather_flat(x_flat, table4, total=total)
    return out4.reshape(x.shape[0], x.shape[1], dim)


# speedup vs baseline: 1.7892x; 1.1342x over previous
import functools

import jax
import jax.numpy as jnp
from jax import lax
from jax.experimental import pallas as pl
from jax.experimental.pallas import tpu as pltpu
from jax.experimental.pallas import tpu_sc as plsc

NUM_CORES = 2
NUM_SUBCORES = 16
NUM_WORKERS = NUM_CORES * NUM_SUBCORES
JB = 16


@functools.partial(jax.jit, static_argnames=("batch", "hist", "dim"))
def _lookup(x, embedding, *, batch, hist, dim):
    rows_per_worker = batch // NUM_WORKERS
    n_blocks = rows_per_worker // JB

    mesh = plsc.VectorSubcoreMesh(
        core_axis_name="c", subcore_axis_name="s",
        num_cores=NUM_CORES, num_subcores=NUM_SUBCORES,
    )

    @functools.partial(
        pl.kernel,
        out_type=jax.ShapeDtypeStruct((batch, hist, dim), jnp.float32),
        mesh=mesh,
        scratch_types=[
            pltpu.VMEM((rows_per_worker, hist), jnp.int32),
            pltpu.VMEM((rows_per_worker * 64,), jnp.int32),
            pltpu.VMEM((JB, hist, dim), jnp.float32),
            pltpu.VMEM((JB, hist, dim), jnp.float32),
            pltpu.SemaphoreType.DMA,
            pltpu.SemaphoreType.DMA,
            pltpu.SemaphoreType.DMA,
            pltpu.SemaphoreType.DMA,
        ],
        compiler_params=pltpu.CompilerParams(use_tc_tiling_on_sc=False,
                                             needs_layout_passes=False),
    )
    def k(x_hbm, table_hbm, out_hbm, idx_v, idx1, ob0, ob1, g0, g1, o0, o1):
        wid = lax.axis_index("s") * NUM_CORES + lax.axis_index("c")
        row0 = wid * rows_per_worker
        pltpu.sync_copy(x_hbm.at[pl.ds(row0, rows_per_worker), :], idx_v)

        obuf = (ob0, ob1)
        gsem = (g0, g1)
        osem = (o0, o1)

        iota = lax.iota(jnp.int32, 16)

        def flat_body(j, carry):
            jvec = j + iota * 0
            for cbase in (0, 16, 32, 48):
                cvec = jnp.minimum(cbase + iota, hist - 1)
                vals = plsc.load_gather(idx_v, [jvec, cvec])
                idx1[pl.ds(j * 64 + cbase, 16)] = vals
            return carry
        lax.fori_loop(0, rows_per_worker, flat_body, 0, unroll=4)

        def gstart(t, b):
            for jj in range(JB):
                pltpu.async_copy(
                    table_hbm.at[idx1.at[pl.ds((t * JB + jj) * 64, hist)]],
                    obuf[b].at[jj], gsem[b])

        def gwait(b):
            for jj in range(JB):
                pltpu.make_async_copy(
                    table_hbm.at[idx1.at[pl.ds(0, hist)]],
                    obuf[b].at[jj], gsem[b]).wait()

        def ostart(t, b):
            pltpu.async_copy(
                obuf[b], out_hbm.at[pl.ds(row0 + t * JB, JB), :, :], osem[b])

        def owait(b):
            pltpu.make_async_copy(
                obuf[b], out_hbm.at[pl.ds(row0, JB), :, :], osem[b]).wait()

        gstart(0, 0)
        gstart(1, 1)

        def pair(t, carry):
            for bb in range(2):
                tt = t + bb
                b = bb
                gwait(b)
                ostart(tt, b)
                owait(b)
                gstart(tt + 2, b)
            return carry
        lax.fori_loop(0, (n_blocks - 2) // 2, lambda p, c: pair(p * 2, c), 0)
        for tt in range(n_blocks - 2, n_blocks):
            b = tt % 2
            gwait(b)
            ostart(tt, b)
            owait(b)

    return k(x, embedding)


def kernel(x, embedding):
    batch, hist = x.shape
    dim = embedding.shape[1]
    return _lookup(x.astype(jnp.int32), embedding,
                   batch=batch, hist=hist, dim=dim)


# speedup vs baseline: 1.7903x; 1.0006x over previous
import functools

import jax
import jax.numpy as jnp
from jax import lax
from jax.experimental import pallas as pl
from jax.experimental.pallas import tpu as pltpu
from jax.experimental.pallas import tpu_sc as plsc

NUM_CORES = 2
NUM_SUBCORES = 16
NUM_WORKERS = NUM_CORES * NUM_SUBCORES
JB = 16


@functools.partial(jax.jit, static_argnames=("batch", "hist", "dim"))
def _lookup(x_flat, embedding, *, batch, hist, dim):
    rows_per_worker = batch // NUM_WORKERS
    n_blocks = rows_per_worker // JB
    lookups_per_worker = rows_per_worker * hist
    out_rows_per_block = JB * hist * dim // 128
    out_rows_per_worker = n_blocks * out_rows_per_block

    mesh = plsc.VectorSubcoreMesh(
        core_axis_name="c", subcore_axis_name="s",
        num_cores=NUM_CORES, num_subcores=NUM_SUBCORES,
    )

    @functools.partial(
        pl.kernel,
        out_type=jax.ShapeDtypeStruct((batch * hist * dim // 128, 128),
                                      jnp.float32),
        mesh=mesh,
        scratch_types=[
            pltpu.VMEM((lookups_per_worker,), jnp.int32),
            pltpu.VMEM((JB * 64,), jnp.int32),
            pltpu.VMEM((JB * 64,), jnp.int32),
            pltpu.VMEM((JB, hist, dim), jnp.float32),
            pltpu.VMEM((JB, hist, dim), jnp.float32),
            pltpu.VMEM((JB * hist * dim // 128, 128), jnp.float32),
            pltpu.SemaphoreType.DMA,
            pltpu.SemaphoreType.DMA,
            pltpu.SemaphoreType.DMA,
        ],
        compiler_params=pltpu.CompilerParams(use_tc_tiling_on_sc=False,
                                             needs_layout_passes=False),
    )
    def k(x_hbm, table_hbm, out_hbm, idx_v, i0, i1, ob0, ob1, ostage,
          g0, g1, osem):
        wid = lax.axis_index("s") * NUM_CORES + lax.axis_index("c")
        base = wid * lookups_per_worker
        out0 = wid * out_rows_per_worker
        pltpu.sync_copy(x_hbm.at[pl.ds(base, lookups_per_worker)], idx_v)

        idx1 = (i0, i1)
        obuf = (ob0, ob1)
        gsem = (g0, g1)
        iota = lax.iota(jnp.int32, 16)

        def flatten(t, b):
            for jj in range(JB):
                for cbase in (0, 16, 32, 48):
                    pos = (t * JB + jj) * hist + jnp.minimum(
                        cbase + iota, hist - 1)
                    idx1[b][pl.ds(jj * 64 + cbase, 16)] = (
                        plsc.load_gather(idx_v, [pos]))

        def gstart(t, b):
            for jj in range(JB):
                pltpu.async_copy(
                    table_hbm.at[idx1[b].at[pl.ds(jj * 64, hist)]],
                    obuf[b].at[jj], gsem[b])

        def gwait(b):
            for jj in range(JB):
                pltpu.make_async_copy(
                    table_hbm.at[idx1[b].at[pl.ds(0, hist)]],
                    obuf[b].at[jj], gsem[b]).wait()

        def repack(b):
            def qbody(q, carry):
                for rr in range(2):
                    for c in range(100):
                        f = 100 * rr + c
                        seg = obuf[b][2 * q + rr, c // 2,
                                      pl.ds((c % 2) * 16, 16)]
                        ostage[25 * q + f // 8,
                               pl.ds((f % 8) * 16, 16)] = seg
                return carry
            lax.fori_loop(0, JB // 2, qbody, 0)

        def ostart(t):
            pltpu.async_copy(
                ostage,
                out_hbm.at[pl.ds(out0 + t * out_rows_per_block,
                                 out_rows_per_block)], osem)

        def owait():
            pltpu.make_async_copy(
                ostage, out_hbm.at[pl.ds(out0, out_rows_per_block)], osem
            ).wait()

        flatten(0, 0)
        gstart(0, 0)
        flatten(1, 1)
        gstart(1, 1)

        for t in (0, 1):
            b = t % 2
            gwait(b)
            if t > 0:
                owait()
            repack(b)
            ostart(t)
            flatten(t + 2, b)
            gstart(t + 2, b)

        def pair(p, carry):
            t0 = p * 2
            for bb in range(2):
                t = t0 + bb
                gwait(bb)
                owait()
                repack(bb)
                ostart(t)
                flatten(t + 2, bb)
                gstart(t + 2, bb)
            return carry
        lax.fori_loop(1, n_blocks // 2 - 1, pair, 0)

        for t in (n_blocks - 2, n_blocks - 1):
            b = t % 2
            gwait(b)
            owait()
            repack(b)
            ostart(t)
        owait()

    return k(x_flat, embedding)


def kernel(x, embedding):
    batch, hist = x.shape
    dim = embedding.shape[1]
    x_flat = jnp.maximum(x.astype(jnp.int32), 0).reshape(-1)
    out4 = _lookup(x_flat, embedding, batch=batch, hist=hist, dim=dim)
    return (out4 + jnp.float32(0.0)).reshape(batch, hist, dim)
